# baseline (speedup 1.0000x reference)
import functools

import jax
import jax.numpy as jnp
from jax import lax
from jax.experimental import pallas as pl
from jax.experimental.pallas import tpu as pltpu
from jax.experimental.pallas import tpu_sc as plsc

B = 8192
D = 1024
BLK = 512
C = B + 4 * BLK
NBLK = C // BLK

NC = 2
NS = 16
NW = NC * NS
L = 16

def _sc_mesh():
    return plsc.VectorSubcoreMesh(core_axis_name="c", subcore_axis_name="s")

_RT_CHUNK = B // NS
_RT_FILL = C // NS

_GX_PER_W = C // NW
_GX_CH = 48
_GX_N = _GX_PER_W // _GX_CH
_UG_PER_W = B // NW
_UG_CH = 32
_UG_N = _UG_PER_W // _UG_CH


def _iota16():
    return lax.iota(jnp.int32, L)


def _route_body(leaf_hbm, pos_hbm, src_hbm, offs_hbm,
                leaf_v, posq_v, tokq_v, zer_v, cnt_v, call_v, offs_v,
                csh, csrc, sem, sem2):
    cid = lax.axis_index("c")
    sid = lax.axis_index("s")

    @pl.when(cid == 0)
    def _work():
        base = sid * _RT_CHUNK
        pltpu.sync_copy(leaf_hbm.at[pl.ds(base, _RT_CHUNK)], leaf_v)
        lane = _iota16()

        acc = [jnp.zeros((L,), jnp.int32) for _ in range(4)]
        for j in range(_RT_CHUNK // L):
            lv = leaf_v[pl.ds(j * L, L)]
            for e in range(4):
                acc[e] += (lv == e).astype(jnp.int32)
        cv = jnp.zeros((L,), jnp.int32)
        for e in range(4):
            cv = jnp.where(lane == e, jnp.sum(acc[e]), cv)
        cnt_v[...] = cv
        pltpu.sync_copy(cnt_v, csh.at[pl.ds(sid * L, L)])

        for j in range(_RT_FILL // L):
            zer_v[pl.ds(j * L, L)] = jnp.zeros((L,), jnp.int32)
        pltpu.sync_copy(zer_v, csrc.at[pl.ds(sid * _RT_FILL, _RT_FILL)])

        plsc.subcore_barrier()

        pltpu.sync_copy(csh, call_v)
        cnt = [[None] * 4 for _ in range(NS)]
        for t in range(NS):
            vt = call_v[pl.ds(t * L, L)]
            for e in range(4):
                cnt[t][e] = jnp.sum(jnp.where(lane == e, vt, 0))
        starts = []
        off_e = jnp.int32(0)
        offs_vec = jnp.zeros((L,), jnp.int32)
        for e in range(4):
            tot = jnp.int32(0)
            before = jnp.int32(0)
            for t in range(NS):
                tot = tot + cnt[t][e]
                before = before + jnp.where(sid > t, cnt[t][e], 0)
            starts.append(off_e + before)
            pad = (tot + (BLK - 1)) & jnp.int32(-BLK)
            offs_vec = offs_vec + jnp.where(lane >= e + 1, pad, 0)
            off_e = off_e + pad

        @pl.when(sid == 0)
        def _write_offs():
            offs_v[...] = offs_vec
            pltpu.sync_copy(offs_v, offs_hbm)

        run = list(starts)
        for j in range(_RT_CHUNK // L):
            lv = leaf_v[pl.ds(j * L, L)]
            posv = jnp.zeros((L,), jnp.int32)
            for e in range(4):
                m = lv == e
                mi = m.astype(jnp.int32)
                posv = jnp.where(m, run[e] + lax.cumsum(mi) - 1, posv)
                run[e] = run[e] + jnp.sum(mi)
            q, r = divmod(j * L, 128)
            posq_v[q, pl.ds(r, L)] = posv
            tokq_v[q, pl.ds(r, L)] = lane + (base + j * L)

        ph = [pltpu.async_copy(posq_v.at[q],
                               pos_hbm.at[pl.ds(base + q * 128, 128)], sem2)
              for q in range(_RT_CHUNK // 128)]
        sh = [pltpu.async_copy(tokq_v.at[q], csrc.at[posq_v.at[q]], sem)
              for q in range(_RT_CHUNK // 128)]
        for h in ph:
            h.wait()
        for h in sh:
            h.wait()
        plsc.subcore_barrier()
        pltpu.sync_copy(csrc.at[pl.ds(sid * _RT_FILL, _RT_FILL)], zer_v)
        pltpu.sync_copy(zer_v, src_hbm.at[pl.ds(sid * _RT_FILL, _RT_FILL)])


def _route(leaf):
    return pl.kernel(
        _route_body,
        out_type=[
            jax.ShapeDtypeStruct((B,), jnp.int32),
            jax.ShapeDtypeStruct((C,), jnp.int32),
            jax.ShapeDtypeStruct((16,), jnp.int32),
        ],
        mesh=_sc_mesh(),
        compiler_params=pltpu.CompilerParams(needs_layout_passes=False),
        scratch_types=[
            pltpu.VMEM((_RT_CHUNK,), jnp.int32),
            pltpu.VMEM((_RT_CHUNK // 128, 128), jnp.int32),
            pltpu.VMEM((_RT_CHUNK // 128, 128), jnp.int32),
            pltpu.VMEM((_RT_FILL,), jnp.int32),
            pltpu.VMEM((L,), jnp.int32),
            pltpu.VMEM((NS * L,), jnp.int32),
            pltpu.VMEM((L,), jnp.int32),
            pltpu.VMEM_SHARED((NS * L,), jnp.int32),
            pltpu.VMEM_SHARED((C,), jnp.int32),
            pltpu.SemaphoreType.DMA,
            pltpu.SemaphoreType.DMA,
        ],
    )(leaf)


def _row_gather(tbl, idx, n_out, chunk, nbuf, dtype, width=D, idx_base=0):
    per_w = n_out // NW
    nch = per_w // chunk

    def body(tbl_hbm, idx_hbm, out_hbm, *scratch):
        idx_v = scratch[0]
        bufs = scratch[1:1 + nbuf]
        sem_g = scratch[1 + nbuf:1 + 2 * nbuf]
        sem_w = scratch[1 + 2 * nbuf:]
        wid = lax.axis_index("s") * NC + lax.axis_index("c")
        base = wid * per_w
        for k in range(nch):
            pltpu.sync_copy(
                idx_hbm.at[pl.ds(idx_base + base + k * chunk, chunk)],
                idx_v.at[k])
        gh = [None] * nbuf
        wh = [None] * nbuf
        for k in range(min(nbuf, nch)):
            gh[k] = pltpu.async_copy(tbl_hbm.at[idx_v.at[k]], bufs[k],
                                     sem_g[k])
        for k in range(nch):
            b = k % nbuf
            gh[b].wait()
            wh[b] = pltpu.async_copy(
                bufs[b], out_hbm.at[pl.ds(base + k * chunk, chunk)], sem_w[b])
            if k + nbuf < nch:
                wh[b].wait()
                gh[b] = pltpu.async_copy(tbl_hbm.at[idx_v.at[k + nbuf]],
                                         bufs[b], sem_g[b])
        for k in range(max(0, nch - nbuf), nch):
            wh[k % nbuf].wait()

    return pl.kernel(
        body,
        out_type=jax.ShapeDtypeStruct((n_out, width), dtype),
        mesh=_sc_mesh(),
        scratch_types=(
            [pltpu.VMEM((nch, chunk), jnp.int32)]
            + [pltpu.VMEM((chunk, width), dtype) for _ in range(nbuf)]
            + [pltpu.SemaphoreType.DMA for _ in range(2 * nbuf)]
        ),
    )(tbl, idx)


_NPART = 2
_PART = C // _NPART


def _gatherx_part(xpack, src, idx_base):
    return _row_gather(xpack, src, _PART, 32, 4, jnp.int32,
                       width=D // 2, idx_base=idx_base)


def _ungather(y_sorted, pos):
    return _row_gather(y_sorted, pos, B, 32, 3, jnp.float32)


_PP_R = 512


def _pack_body(x_ref, o_ref):
    v = x_ref[...].astype(jnp.bfloat16)
    o_ref[...] = pltpu.bitcast(v.reshape(2 * _PP_R, D // 2), jnp.int32)


def _pack_x(x):
    return pl.pallas_call(
        _pack_body,
        grid=(B // _PP_R,),
        in_specs=[pl.BlockSpec((_PP_R, D), lambda i: (i, 0))],
        out_specs=pl.BlockSpec((_PP_R, D // 2), lambda i: (i, 0)),
        out_shape=jax.ShapeDtypeStruct((B, D // 2), jnp.int32),
    )(x)


def _compose_d1_body(wr_ref, w1_ref, br_ref, b1_ref, t_ref, bt_ref):
    w1 = w1_ref[0]
    t_ref[0] = jnp.dot(wr_ref[...], w1, preferred_element_type=jnp.float32)
    bt_ref[0] = jnp.dot(br_ref[...], w1, preferred_element_type=jnp.float32) + b1_ref[0]


def _compose_d2_body(t_ref, w2_ref, bt_ref, b2_ref, wc_ref, bc_ref):
    w2 = w2_ref[0]
    wc = jnp.dot(t_ref[0], w2, preferred_element_type=jnp.float32)
    wc_ref[0] = wc.astype(jnp.bfloat16)
    bc_ref[0] = jnp.dot(bt_ref[0], w2, preferred_element_type=jnp.float32) + b2_ref[0]


def _compose(W_root, b_root, W_d1, b_d1, W_d2, b_d2):
    br = b_root.reshape(1, D)
    b1 = b_d1.reshape(2, 1, D)
    b2 = b_d2.reshape(4, 1, D)
    T, bt = pl.pallas_call(
        _compose_d1_body,
        grid=(2,),
        in_specs=[
            pl.BlockSpec((D, D), lambda c: (0, 0)),
            pl.BlockSpec((1, D, D), lambda c: (c, 0, 0)),
            pl.BlockSpec((1, D), lambda c: (0, 0)),
            pl.BlockSpec((1, 1, D), lambda c: (c, 0, 0)),
        ],
        out_specs=[
            pl.BlockSpec((1, D, D), lambda c: (c, 0, 0)),
            pl.BlockSpec((1, 1, D), lambda c: (c, 0, 0)),
        ],
        out_shape=[
            jax.ShapeDtypeStruct((2, D, D), jnp.float32),
            jax.ShapeDtypeStruct((2, 1, D), jnp.float32),
        ],
    )(W_root, W_d1, br, b1)
    Wc, bc = pl.pallas_call(
        _compose_d2_body,
        grid=(4,),
        in_specs=[
            pl.BlockSpec((1, D, D), lambda e: (e // 2, 0, 0)),
            pl.BlockSpec((1, D, D), lambda e: (e, 0, 0)),
            pl.BlockSpec((1, 1, D), lambda e: (e // 2, 0, 0)),
            pl.BlockSpec((1, 1, D), lambda e: (e, 0, 0)),
        ],
        out_specs=[
            pl.BlockSpec((1, D, D), lambda e: (e, 0, 0)),
            pl.BlockSpec((1, 1, D), lambda e: (e, 0, 0)),
        ],
        out_shape=[
            jax.ShapeDtypeStruct((4, D, D), jnp.bfloat16),
            jax.ShapeDtypeStruct((4, 1, D), jnp.float32),
        ],
    )(T, W_d2, bt, b2)
    return Wc, bc


def _routed_mm_body(off_ref, x_ref, wc_ref, bc_ref, o_ref):
    del off_ref
    xb = pltpu.bitcast(x_ref[...], jnp.bfloat16).reshape(BLK, D)
    o_ref[...] = (
        jnp.dot(xb, wc_ref[0], preferred_element_type=jnp.float32)
        + bc_ref[0]
    )


def _routed_mm_body_alias(off_ref, x_ref, wc_ref, bc_ref, yprev_ref, o_ref):
    del off_ref, yprev_ref
    xb = pltpu.bitcast(x_ref[...], jnp.bfloat16).reshape(BLK, D)
    o_ref[...] = (
        jnp.dot(xb, wc_ref[0], preferred_element_type=jnp.float32)
        + bc_ref[0]
    )


def _block_expert(b, off_ref):
    s = b * BLK
    return (
        (s >= off_ref[1]).astype(jnp.int32)
        + (s >= off_ref[2]).astype(jnp.int32)
        + (s >= off_ref[3]).astype(jnp.int32)
    )


def _routed_mm_half(off, x_half, Wc, bc, b0, y_prev=None):
    nb = x_half.shape[0] // BLK
    in_specs = [
        pl.BlockSpec((BLK, D // 2), lambda b, off_ref: (b, 0)),
        pl.BlockSpec((1, D, D),
                     lambda b, off_ref: (_block_expert(b + b0, off_ref), 0, 0)),
        pl.BlockSpec((1, 1, D),
                     lambda b, off_ref: (_block_expert(b + b0, off_ref), 0, 0)),
    ]
    args = [off, x_half, Wc, bc]
    io_alias = {}
    body = _routed_mm_body
    if y_prev is not None:
        in_specs.append(pl.BlockSpec(memory_space=pl.ANY))
        args.append(y_prev)
        io_alias = {4: 0}
        body = _routed_mm_body_alias
    spec = pltpu.PrefetchScalarGridSpec(
        num_scalar_prefetch=1,
        grid=(nb,),
        in_specs=in_specs,
        out_specs=pl.BlockSpec((BLK, D), lambda b, off_ref: (b + b0, 0)),
    )
    return pl.pallas_call(
        body,
        grid_spec=spec,
        out_shape=jax.ShapeDtypeStruct((C, D), jnp.float32),
        input_output_aliases=io_alias,
    )(*args)


def kernel(x, path_mask, W_root, b_root, W_d1, b_d1, W_d2, b_d2):
    leaf = path_mask[:, 0] * 2 + path_mask[:, 1]
    pos, src, offs = _route(leaf)
    Wc, bc = _compose(W_root, b_root, W_d1, b_d1, W_d2, b_d2)
    xpack = _pack_x(x)
    xs = [_gatherx_part(xpack, src, q * _PART) for q in range(_NPART)]
    y = None
    for q in range(_NPART):
        y = _routed_mm_half(offs, xs[q], Wc, bc, q * (_PART // BLK),
                            y_prev=y)
    return _ungather(y, pos)


# speedup vs baseline: 1.2092x; 1.2092x over previous
import functools

import jax
import jax.numpy as jnp
from jax import lax
from jax.experimental import pallas as pl
from jax.experimental.pallas import tpu as pltpu
from jax.experimental.pallas import tpu_sc as plsc

B = 8192
D = 1024
BLK = 256
C = B + 4 * BLK
NBLK = C // BLK

NC = 2
NS = 16
NW = NC * NS
L = 16

def _sc_mesh():
    return plsc.VectorSubcoreMesh(core_axis_name="c", subcore_axis_name="s")

_RT_CHUNK = B // NS
_RT_FILL = C // NS

_GX_PER_W = C // NW
_GX_CH = 48
_GX_N = _GX_PER_W // _GX_CH
_UG_PER_W = B // NW
_UG_CH = 32
_UG_N = _UG_PER_W // _UG_CH


def _iota16():
    return lax.iota(jnp.int32, L)


def _route_body(leaf_hbm, pos_hbm, src_hbm, offs_hbm,
                leaf_v, posq_v, tokq_v, zer_v, cnt_v, call_v, offs_v,
                csh, csrc, sem, sem2):
    cid = lax.axis_index("c")
    sid = lax.axis_index("s")

    @pl.when(cid == 0)
    def _work():
        base = sid * _RT_CHUNK
        pltpu.sync_copy(leaf_hbm.at[pl.ds(base, _RT_CHUNK)], leaf_v)
        lane = _iota16()

        acc = [jnp.zeros((L,), jnp.int32) for _ in range(4)]
        for j in range(_RT_CHUNK // L):
            lv = leaf_v[pl.ds(j * L, L)]
            for e in range(4):
                acc[e] += (lv == e).astype(jnp.int32)
        cv = jnp.zeros((L,), jnp.int32)
        for e in range(4):
            cv = jnp.where(lane == e, jnp.sum(acc[e]), cv)
        cnt_v[...] = cv
        pltpu.sync_copy(cnt_v, csh.at[pl.ds(sid * L, L)])

        for j in range(_RT_FILL // L):
            zer_v[pl.ds(j * L, L)] = jnp.zeros((L,), jnp.int32)
        pltpu.sync_copy(zer_v, csrc.at[pl.ds(sid * _RT_FILL, _RT_FILL)])

        plsc.subcore_barrier()

        pltpu.sync_copy(csh, call_v)
        cnt = [[None] * 4 for _ in range(NS)]
        for t in range(NS):
            vt = call_v[pl.ds(t * L, L)]
            for e in range(4):
                cnt[t][e] = jnp.sum(jnp.where(lane == e, vt, 0))
        starts = []
        off_e = jnp.int32(0)
        offs_vec = jnp.zeros((L,), jnp.int32)
        for e in range(4):
            tot = jnp.int32(0)
            before = jnp.int32(0)
            for t in range(NS):
                tot = tot + cnt[t][e]
                before = before + jnp.where(sid > t, cnt[t][e], 0)
            starts.append(off_e + before)
            pad = (tot + (BLK - 1)) & jnp.int32(-BLK)
            offs_vec = offs_vec + jnp.where(lane >= e + 1, pad, 0)
            off_e = off_e + pad

        @pl.when(sid == 0)
        def _write_offs():
            offs_v[...] = offs_vec
            pltpu.sync_copy(offs_v, offs_hbm)

        run = list(starts)
        for j in range(_RT_CHUNK // L):
            lv = leaf_v[pl.ds(j * L, L)]
            posv = jnp.zeros((L,), jnp.int32)
            for e in range(4):
                m = lv == e
                mi = m.astype(jnp.int32)
                posv = jnp.where(m, run[e] + lax.cumsum(mi) - 1, posv)
                run[e] = run[e] + jnp.sum(mi)
            q, r = divmod(j * L, 128)
            posq_v[q, pl.ds(r, L)] = posv
            tokq_v[q, pl.ds(r, L)] = lane + (base + j * L)

        ph = [pltpu.async_copy(posq_v.at[q],
                               pos_hbm.at[pl.ds(base + q * 128, 128)], sem2)
              for q in range(_RT_CHUNK // 128)]
        sh = [pltpu.async_copy(tokq_v.at[q], csrc.at[posq_v.at[q]], sem)
              for q in range(_RT_CHUNK // 128)]
        for h in ph:
            h.wait()
        for h in sh:
            h.wait()
        plsc.subcore_barrier()
        pltpu.sync_copy(csrc.at[pl.ds(sid * _RT_FILL, _RT_FILL)], zer_v)
        pltpu.sync_copy(zer_v, src_hbm.at[pl.ds(sid * _RT_FILL, _RT_FILL)])


def _route(leaf):
    return pl.kernel(
        _route_body,
        out_type=[
            jax.ShapeDtypeStruct((B,), jnp.int32),
            jax.ShapeDtypeStruct((C,), jnp.int32),
            jax.ShapeDtypeStruct((16,), jnp.int32),
        ],
        mesh=_sc_mesh(),
        compiler_params=pltpu.CompilerParams(needs_layout_passes=False),
        scratch_types=[
            pltpu.VMEM((_RT_CHUNK,), jnp.int32),
            pltpu.VMEM((_RT_CHUNK // 128, 128), jnp.int32),
            pltpu.VMEM((_RT_CHUNK // 128, 128), jnp.int32),
            pltpu.VMEM((_RT_FILL,), jnp.int32),
            pltpu.VMEM((L,), jnp.int32),
            pltpu.VMEM((NS * L,), jnp.int32),
            pltpu.VMEM((L,), jnp.int32),
            pltpu.VMEM_SHARED((NS * L,), jnp.int32),
            pltpu.VMEM_SHARED((C,), jnp.int32),
            pltpu.SemaphoreType.DMA,
            pltpu.SemaphoreType.DMA,
        ],
    )(leaf)


def _row_gather(tbl, idx, n_out, chunk, nbuf, dtype, width=D, idx_base=0):
    per_w = n_out // NW
    nch = per_w // chunk

    def body(tbl_hbm, idx_hbm, out_hbm, *scratch):
        idx_v = scratch[0]
        bufs = scratch[1:1 + nbuf]
        sem_g = scratch[1 + nbuf:1 + 2 * nbuf]
        sem_w = scratch[1 + 2 * nbuf:]
        wid = lax.axis_index("s") * NC + lax.axis_index("c")
        base = wid * per_w
        for k in range(nch):
            pltpu.sync_copy(
                idx_hbm.at[pl.ds(idx_base + base + k * chunk, chunk)],
                idx_v.at[k])
        gh = [None] * nbuf
        wh = [None] * nbuf
        for k in range(min(nbuf, nch)):
            gh[k] = pltpu.async_copy(tbl_hbm.at[idx_v.at[k]], bufs[k],
                                     sem_g[k])
        for k in range(nch):
            b = k % nbuf
            gh[b].wait()
            wh[b] = pltpu.async_copy(
                bufs[b], out_hbm.at[pl.ds(base + k * chunk, chunk)], sem_w[b])
            if k + nbuf < nch:
                wh[b].wait()
                gh[b] = pltpu.async_copy(tbl_hbm.at[idx_v.at[k + nbuf]],
                                         bufs[b], sem_g[b])
        for k in range(max(0, nch - nbuf), nch):
            wh[k % nbuf].wait()

    return pl.kernel(
        body,
        out_type=jax.ShapeDtypeStruct((n_out, width), dtype),
        mesh=_sc_mesh(),
        scratch_types=(
            [pltpu.VMEM((nch, chunk), jnp.int32)]
            + [pltpu.VMEM((chunk, width), dtype) for _ in range(nbuf)]
            + [pltpu.SemaphoreType.DMA for _ in range(2 * nbuf)]
        ),
    )(tbl, idx)


_NPART = 3
_PART = C // _NPART


def _gatherx_part(xpack, src, idx_base):
    return _row_gather(xpack, src, _PART, 24, 4, jnp.int32,
                       width=D // 2, idx_base=idx_base)


def _ungather(y_sorted, pos):
    return _row_gather(y_sorted, pos, B, 32, 3, jnp.float32)


_PP_R = 512


def _pack_body(x_ref, o_ref):
    v = x_ref[...].astype(jnp.bfloat16)
    o_ref[...] = pltpu.bitcast(v.reshape(2 * _PP_R, D // 2), jnp.int32)


def _pack_x(x):
    return pl.pallas_call(
        _pack_body,
        grid=(B // _PP_R,),
        in_specs=[pl.BlockSpec((_PP_R, D), lambda i: (i, 0))],
        out_specs=pl.BlockSpec((_PP_R, D // 2), lambda i: (i, 0)),
        out_shape=jax.ShapeDtypeStruct((B, D // 2), jnp.int32),
    )(x)


def _compose_d1_body(wr_ref, w1_ref, br_ref, b1_ref, t_ref, bt_ref):
    w1 = w1_ref[0]
    t_ref[0] = jnp.dot(wr_ref[...], w1, preferred_element_type=jnp.float32)
    bt_ref[0] = jnp.dot(br_ref[...], w1, preferred_element_type=jnp.float32) + b1_ref[0]


def _compose_d2_body(t_ref, w2_ref, bt_ref, b2_ref, wc_ref, bc_ref):
    w2 = w2_ref[0]
    wc = jnp.dot(t_ref[0], w2, preferred_element_type=jnp.float32)
    wc_ref[0] = wc.astype(jnp.bfloat16)
    bc_ref[0] = jnp.dot(bt_ref[0], w2, preferred_element_type=jnp.float32) + b2_ref[0]


def _compose(W_root, b_root, W_d1, b_d1, W_d2, b_d2):
    br = b_root.reshape(1, D)
    b1 = b_d1.reshape(2, 1, D)
    b2 = b_d2.reshape(4, 1, D)
    T, bt = pl.pallas_call(
        _compose_d1_body,
        grid=(2,),
        in_specs=[
            pl.BlockSpec((D, D), lambda c: (0, 0)),
            pl.BlockSpec((1, D, D), lambda c: (c, 0, 0)),
            pl.BlockSpec((1, D), lambda c: (0, 0)),
            pl.BlockSpec((1, 1, D), lambda c: (c, 0, 0)),
        ],
        out_specs=[
            pl.BlockSpec((1, D, D), lambda c: (c, 0, 0)),
            pl.BlockSpec((1, 1, D), lambda c: (c, 0, 0)),
        ],
        out_shape=[
            jax.ShapeDtypeStruct((2, D, D), jnp.float32),
            jax.ShapeDtypeStruct((2, 1, D), jnp.float32),
        ],
    )(W_root, W_d1, br, b1)
    Wc, bc = pl.pallas_call(
        _compose_d2_body,
        grid=(4,),
        in_specs=[
            pl.BlockSpec((1, D, D), lambda e: (e // 2, 0, 0)),
            pl.BlockSpec((1, D, D), lambda e: (e, 0, 0)),
            pl.BlockSpec((1, 1, D), lambda e: (e // 2, 0, 0)),
            pl.BlockSpec((1, 1, D), lambda e: (e, 0, 0)),
        ],
        out_specs=[
            pl.BlockSpec((1, D, D), lambda e: (e, 0, 0)),
            pl.BlockSpec((1, 1, D), lambda e: (e, 0, 0)),
        ],
        out_shape=[
            jax.ShapeDtypeStruct((4, D, D), jnp.bfloat16),
            jax.ShapeDtypeStruct((4, 1, D), jnp.float32),
        ],
    )(T, W_d2, bt, b2)
    return Wc, bc


def _routed_mm_body(off_ref, x_ref, wc_ref, bc_ref, o_ref):
    del off_ref
    xb = pltpu.bitcast(x_ref[...], jnp.bfloat16).reshape(BLK, D)
    o_ref[...] = (
        jnp.dot(xb, wc_ref[0], preferred_element_type=jnp.float32)
        + bc_ref[0]
    )


def _routed_mm_body_alias(off_ref, x_ref, wc_ref, bc_ref, yprev_ref, o_ref):
    del off_ref, yprev_ref
    xb = pltpu.bitcast(x_ref[...], jnp.bfloat16).reshape(BLK, D)
    o_ref[...] = (
        jnp.dot(xb, wc_ref[0], preferred_element_type=jnp.float32)
        + bc_ref[0]
    )


def _block_expert(b, off_ref):
    s = b * BLK
    return (
        (s >= off_ref[1]).astype(jnp.int32)
        + (s >= off_ref[2]).astype(jnp.int32)
        + (s >= off_ref[3]).astype(jnp.int32)
    )


def _routed_mm_half(off, x_half, Wc, bc, b0, y_prev=None):
    nb = x_half.shape[0] // BLK
    in_specs = [
        pl.BlockSpec((BLK, D // 2), lambda b, off_ref: (b, 0)),
        pl.BlockSpec((1, D, D),
                     lambda b, off_ref: (_block_expert(b + b0, off_ref), 0, 0)),
        pl.BlockSpec((1, 1, D),
                     lambda b, off_ref: (_block_expert(b + b0, off_ref), 0, 0)),
    ]
    args = [off, x_half, Wc, bc]
    io_alias = {}
    body = _routed_mm_body
    if y_prev is not None:
        in_specs.append(pl.BlockSpec(memory_space=pl.ANY))
        args.append(y_prev)
        io_alias = {4: 0}
        body = _routed_mm_body_alias
    spec = pltpu.PrefetchScalarGridSpec(
        num_scalar_prefetch=1,
        grid=(nb,),
        in_specs=in_specs,
        out_specs=pl.BlockSpec((BLK, D), lambda b, off_ref: (b + b0, 0)),
    )
    return pl.pallas_call(
        body,
        grid_spec=spec,
        out_shape=jax.ShapeDtypeStruct((C, D), jnp.float32),
        input_output_aliases=io_alias,
    )(*args)


def kernel(x, path_mask, W_root, b_root, W_d1, b_d1, W_d2, b_d2):
    leaf = path_mask[:, 0] * 2 + path_mask[:, 1]
    pos, src, offs = _route(leaf)
    Wc, bc = _compose(W_root, b_root, W_d1, b_d1, W_d2, b_d2)
    xpack = _pack_x(x)
    xs = [_gatherx_part(xpack, src, q * _PART) for q in range(_NPART)]
    y = None
    for q in range(_NPART):
        y = _routed_mm_half(offs, xs[q], Wc, bc, q * (_PART // BLK),
                            y_prev=y)
    return _ungather(y, pos)
